# baseline (speedup 1.0000x reference)
import functools

import jax
import jax.numpy as jnp
import numpy as np
from jax import lax
from jax.experimental import pallas as pl
from jax.experimental.pallas import tpu as pltpu
from jax.experimental.pallas import tpu_sc as plsc

_N_LEVELS = 16
_F = 2
_LOG2_T = 19
_T = 1 << _LOG2_T
_MASK = _T - 1
_P1 = int(np.uint32(2654435761).view(np.int32))
_P2 = 805459861

_FACTOR = np.exp((np.log(512.0) - np.log(16.0)) / (_N_LEVELS - 1))
_GS = [
    float(np.float32(2.0 / float(np.floor(16.0 * _FACTOR**i))))
    for i in range(_N_LEVELS)
]

_N_GRID = 6
_C = 256
_W = 8 * _C


def _grid_consts():
    out = []
    for lvl in range(_N_GRID):
        gs = np.float32(_GS[lvl])
        bmin = int(np.floor(np.float32(1.0) / gs))
        bmax = int(np.floor(np.float32(2.0) / gs))
        side = bmax - bmin + 2
        nv = side * side * side
        nvp = ((nv + 15) // 16) * 16
        out.append((bmin, side, nvp))
    return out


_GRIDS = _grid_consts()
_GPAD = [
    ((2 * nvp + 2 * _W - 1) // (2 * _W)) * (2 * _W) for (_, _, nvp) in _GRIDS
]
_GOFF = [sum(_GPAD[:i]) for i in range(_N_GRID)]
_GTOT = sum(_GPAD)
_PIECES = [
    (lvl, pc, _GOFF[lvl] + pc * 2 * _W)
    for lvl in range(_N_GRID)
    for pc in range(_GPAD[lvl] // (2 * _W))
]


def _sc_info():
    try:
        info = plsc.get_sparse_core_info()
        return info.num_cores, info.num_subcores
    except Exception:
        return 2, 16


@functools.lru_cache(maxsize=None)
def _build_grid_kernel():
    nc, ns = _sc_info()
    mesh = plsc.VectorSubcoreMesh(
        core_axis_name="c", subcore_axis_name="s", num_cores=nc, num_subcores=ns
    )

    def body(tabh, gridsh, idxb, valb, sem):
        wid = lax.axis_index("c") * ns + lax.axis_index("s")
        iota = lax.iota(jnp.int32, 16)

        for pid, (lvl, pc, goff) in enumerate(_PIECES):
            bmin, side, nvp = _GRIDS[lvl]
            s2 = side * side
            ioff = lvl * _T * _F

            @pl.when(wid == pid)
            def _(s2=s2, side=side, bmin=bmin, ioff=ioff, nvp=nvp, pc=pc,
                  goff=goff):
                def b_body(g, _):
                    pos = pc * (2 * _W) + g * 16
                    fflag = jnp.where(pos >= nvp, jnp.int32(1), jnp.int32(0))
                    v = pos - fflag * nvp + iota
                    a = v // jnp.int32(s2)
                    r1 = v - a * jnp.int32(s2)
                    b = r1 // jnp.int32(side)
                    c = r1 - b * jnp.int32(side)
                    h = ((a + jnp.int32(bmin))
                         ^ ((b + jnp.int32(bmin)) * jnp.int32(_P1))
                         ^ ((c + jnp.int32(bmin)) * jnp.int32(_P2)))
                    hm = h & jnp.int32(_MASK)
                    t = hm & jnp.int32(127)
                    p0 = ((hm - t) << 1) + t + jnp.int32(ioff)
                    idxb[pl.ds(g * 16, 16)] = p0 + fflag * 128
                    return 0

                lax.fori_loop(0, (2 * _W) // 16, b_body, 0)
                pltpu.async_copy(tabh.at[idxb], valb, sem).wait()
                pltpu.sync_copy(valb, gridsh.at[pl.ds(goff, 2 * _W)])

    return pl.kernel(
        body,
        out_type=jax.ShapeDtypeStruct((_GTOT,), jnp.float32),
        mesh=mesh,
        compiler_params=pltpu.CompilerParams(needs_layout_passes=False),
        scratch_types=[
            pltpu.VMEM((2 * _W,), jnp.int32),
            pltpu.VMEM((2 * _W,), jnp.float32),
            pltpu.SemaphoreType.DMA,
        ],
    )


@functools.lru_cache(maxsize=None)
def _build(n):
    nc, ns = _sc_info()
    nw = nc * ns
    pw = n // nw
    nchunk = pw // _C
    nf = _N_LEVELS * _F
    mesh = plsc.VectorSubcoreMesh(
        core_axis_name="c", subcore_axis_name="s", num_cores=nc, num_subcores=ns
    )

    def body(xh, tabh, gridsh, outh, xb, wb, idxb0, idxb1, rowsb0, rowsb1,
             outb, g0, g1, g2, g3, g4, g5, sem0, sem1, osem):
        wid = lax.axis_index("c") * ns + lax.axis_index("s")
        iota = lax.iota(jnp.int32, 16)
        lane_nf = iota * nf
        lane3 = iota * 3
        idxbs = (idxb0, idxb1)
        rowsbs = (rowsb0, rowsb1)
        sems = (sem0, sem1)
        grids = (g0, g1, g2, g3, g4, g5)

        def cell(xv, gs):
            q = (xv + 1.0) / gs
            bi = q.astype(jnp.int32)
            bf = bi.astype(jnp.float32)
            minv = bf * gs - 1.0
            maxv = minv + gs
            w = (xv - minv) / (maxv - minv)
            return bi, w

        def hash_words(gx, gy, gz, ioff):
            h = gx ^ (gy * jnp.int32(_P1)) ^ (gz * jnp.int32(_P2))
            hm = h & jnp.int32(_MASK)
            t = hm & jnp.int32(127)
            return ((hm - t) << 1) + t + jnp.int32(ioff)

        def build_grids():
            for lvl in range(_N_GRID):
                pltpu.sync_copy(
                    gridsh.at[pl.ds(_GOFF[lvl], _GPAD[lvl])], grids[lvl]
                )

        def idx_pass(lvl, par):
            gs = _GS[lvl]
            ioff = lvl * _T * _F

            def g_body(g, _):
                o = g * 16
                cs = []
                for d in range(3):
                    xv = plsc.load_gather(xb, [lane3 + (o * 3 + d)])
                    bi, w = cell(xv, gs)
                    wb[par, d, pl.ds(o, 16)] = w
                    cs.append(bi)
                c0, c1, c2 = cs
                c1p = c1 * jnp.int32(_P1)
                c2p = c2 * jnp.int32(_P2)
                combos = ((c0, c0 + jnp.int32(1)),
                          (c1p, c1p + jnp.int32(_P1)),
                          (c2p, c2p + jnp.int32(_P2)))
                for j in range(8):
                    h = (combos[0][(j >> 2) & 1] ^ combos[1][(j >> 1) & 1]
                         ^ combos[2][j & 1])
                    hm = h & jnp.int32(_MASK)
                    t = hm & jnp.int32(127)
                    p0 = ((hm - t) << 1) + t + jnp.int32(ioff)
                    idxbs[par][pl.ds(j * _C + o, 16)] = p0
                    idxbs[par][pl.ds(_W + j * _C + o, 16)] = p0 + 128
                return 0

            lax.fori_loop(0, _C // 16, g_body, 0)

        def start_gather(par):
            return pltpu.async_copy(
                tabh.at[idxbs[par]], rowsbs[par], sems[par]
            )

        def trilerp_store(v, w0, w1, w2, o, lvl):
            for f in range(2):
                m = v[f]
                m = [m[2 * a] + w2 * (m[2 * a + 1] - m[2 * a])
                     for a in range(4)]
                m = [m[2 * a] + w1 * (m[2 * a + 1] - m[2 * a])
                     for a in range(2)]
                r = m[0] + w0 * (m[1] - m[0])
                plsc.store_scatter(
                    outb, [lane_nf + (o * nf + 2 * lvl + f)], r
                )

        def interp_pass(lvl, par):
            def g_body(g, _):
                o = g * 16
                w0 = wb[par, 0, pl.ds(o, 16)]
                w1 = wb[par, 1, pl.ds(o, 16)]
                w2 = wb[par, 2, pl.ds(o, 16)]
                v = [[rowsbs[par][pl.ds(f * _W + j * _C + o, 16)]
                      for j in range(8)] for f in range(2)]
                trilerp_store(v, w0, w1, w2, o, lvl)
                return 0

            lax.fori_loop(0, _C // 16, g_body, 0)

        def grid_pass(lvl):
            bmin, side, nvp = _GRIDS[lvl]
            s2 = side * side
            gs = _GS[lvl]
            gref = grids[lvl]
            lvbase = bmin * (s2 + side + 1)

            def g_body(g, _):
                o = g * 16
                bis, ws = [], []
                for d in range(3):
                    xv = plsc.load_gather(xb, [lane3 + (o * 3 + d)])
                    bi, w = cell(xv, gs)
                    bis.append(bi)
                    ws.append(w)
                lv = (bis[0] * jnp.int32(s2) + bis[1] * jnp.int32(side)
                      + bis[2] - jnp.int32(lvbase))
                v = [[None] * 8, [None] * 8]
                for j in range(8):
                    cj = (((j >> 2) & 1) * s2 + ((j >> 1) & 1) * side
                          + (j & 1))
                    v[0][j] = plsc.load_gather(gref, [lv + jnp.int32(cj)])
                    v[1][j] = plsc.load_gather(
                        gref, [lv + jnp.int32(cj + nvp)]
                    )
                trilerp_store(v, ws[0], ws[1], ws[2], o, lvl)
                return 0

            lax.fori_loop(0, _C // 16, g_body, 0)

        build_grids()

        def chunk_body(c, _):
            base = wid * pw + c * _C
            pltpu.sync_copy(xh.at[pl.ds(base * 3, _C * 3)], xb)
            idx_pass(_N_GRID, 0)
            dmas = [start_gather(0), None]
            idx_pass(_N_GRID + 1, 1)
            dmas[1] = start_gather(1)

            @pl.when(c > 0)
            def _():
                pltpu.make_async_copy(
                    outb, outh.at[pl.ds(wid * pw * nf, _C * nf)], osem
                ).wait()

            for lvl in range(_N_GRID):
                grid_pass(lvl)
            for lvl in range(_N_GRID, _N_LEVELS):
                par = (lvl - _N_GRID) & 1
                dmas[par].wait()
                interp_pass(lvl, par)
                if lvl + 2 < _N_LEVELS:
                    idx_pass(lvl + 2, par)
                    dmas[par] = start_gather(par)
            pltpu.async_copy(outb, outh.at[pl.ds(base * nf, _C * nf)], osem)
            return 0

        lax.fori_loop(0, nchunk, chunk_body, 0)
        pltpu.make_async_copy(
            outb, outh.at[pl.ds(wid * pw * nf, _C * nf)], osem
        ).wait()

    grid_scratch = [pltpu.VMEM((gp,), jnp.float32) for gp in _GPAD]
    return pl.kernel(
        body,
        out_type=jax.ShapeDtypeStruct((n * nf,), jnp.float32),
        mesh=mesh,
        compiler_params=pltpu.CompilerParams(needs_layout_passes=False),
        scratch_types=[
            pltpu.VMEM((3 * _C,), jnp.float32),
            pltpu.VMEM((2, 3, _C), jnp.float32),
            pltpu.VMEM((2 * _W,), jnp.int32),
            pltpu.VMEM((2 * _W,), jnp.int32),
            pltpu.VMEM((2 * _W,), jnp.float32),
            pltpu.VMEM((2 * _W,), jnp.float32),
            pltpu.VMEM((_C * nf,), jnp.float32),
            *grid_scratch,
            pltpu.SemaphoreType.DMA,
            pltpu.SemaphoreType.DMA,
            pltpu.SemaphoreType.DMA,
        ],
    )


def kernel(x, tables):
    n = x.shape[0]
    xf = x.reshape(n * 3)
    tab = (
        tables.reshape(_N_LEVELS, _T // 128, 128, _F)
        .transpose(0, 1, 3, 2)
        .reshape(_N_LEVELS * _T * _F)
    )
    grids = _build_grid_kernel()(tab)
    out = _build(n)(xf, tab, grids)
    return out.reshape(n, _N_LEVELS * _F)


# speedup vs baseline: 1.0741x; 1.0741x over previous
import functools

import jax
import jax.numpy as jnp
import numpy as np
from jax import lax
from jax.experimental import pallas as pl
from jax.experimental.pallas import tpu as pltpu
from jax.experimental.pallas import tpu_sc as plsc

_N_LEVELS = 16
_F = 2
_LOG2_T = 19
_T = 1 << _LOG2_T
_MASK = _T - 1
_P1 = int(np.uint32(2654435761).view(np.int32))
_P2 = 805459861

_FACTOR = np.exp((np.log(512.0) - np.log(16.0)) / (_N_LEVELS - 1))
_GS = [
    float(np.float32(2.0 / float(np.floor(16.0 * _FACTOR**i))))
    for i in range(_N_LEVELS)
]

_N_GRID = 6
_C = 256
_W = 8 * _C


def _grid_consts():
    out = []
    for lvl in range(_N_GRID):
        gs = np.float32(_GS[lvl])
        bmin = int(np.floor(np.float32(1.0) / gs))
        bmax = int(np.floor(np.float32(2.0) / gs))
        side = bmax - bmin + 2
        nv = side * side * side
        nvp = ((nv + 15) // 16) * 16
        out.append((bmin, side, nvp))
    return out


_GRIDS = _grid_consts()
_GPAD = [
    ((2 * nvp + 2 * _W - 1) // (2 * _W)) * (2 * _W) for (_, _, nvp) in _GRIDS
]
_GOFF = [sum(_GPAD[:i]) for i in range(_N_GRID)]
_GTOT = sum(_GPAD)
_PIECES = [
    (lvl, pc, _GOFF[lvl] + pc * 2 * _W)
    for lvl in range(_N_GRID)
    for pc in range(_GPAD[lvl] // (2 * _W))
]


def _sc_info():
    try:
        info = plsc.get_sparse_core_info()
        return info.num_cores, info.num_subcores
    except Exception:
        return 2, 16


@functools.lru_cache(maxsize=None)
def _build_grid_kernel():
    nc, ns = _sc_info()
    mesh = plsc.VectorSubcoreMesh(
        core_axis_name="c", subcore_axis_name="s", num_cores=nc, num_subcores=ns
    )

    def body(tabh, gridsh, idxb, valb, sem):
        wid = lax.axis_index("c") * ns + lax.axis_index("s")
        iota = lax.iota(jnp.int32, 16)

        for pid, (lvl, pc, goff) in enumerate(_PIECES):
            bmin, side, nvp = _GRIDS[lvl]
            s2 = side * side
            ioff = lvl * _T * _F

            @pl.when(wid == pid)
            def _(s2=s2, side=side, bmin=bmin, ioff=ioff, nvp=nvp, pc=pc,
                  goff=goff):
                def b_body(g, _):
                    pos = pc * (2 * _W) + g * 16
                    fflag = jnp.where(pos >= nvp, jnp.int32(1), jnp.int32(0))
                    v = pos - fflag * nvp + iota
                    a = v // jnp.int32(s2)
                    r1 = v - a * jnp.int32(s2)
                    b = r1 // jnp.int32(side)
                    c = r1 - b * jnp.int32(side)
                    h = ((a + jnp.int32(bmin))
                         ^ ((b + jnp.int32(bmin)) * jnp.int32(_P1))
                         ^ ((c + jnp.int32(bmin)) * jnp.int32(_P2)))
                    hm = h & jnp.int32(_MASK)
                    t = hm & jnp.int32(127)
                    p0 = ((hm - t) << 1) + t + jnp.int32(ioff)
                    idxb[pl.ds(g * 16, 16)] = p0 + fflag * 128
                    return 0

                lax.fori_loop(0, (2 * _W) // 16, b_body, 0)
                pltpu.async_copy(tabh.at[idxb], valb, sem).wait()
                pltpu.sync_copy(valb, gridsh.at[pl.ds(goff, 2 * _W)])

    return pl.kernel(
        body,
        out_type=jax.ShapeDtypeStruct((_GTOT,), jnp.float32),
        mesh=mesh,
        compiler_params=pltpu.CompilerParams(needs_layout_passes=False),
        scratch_types=[
            pltpu.VMEM((2 * _W,), jnp.int32),
            pltpu.VMEM((2 * _W,), jnp.float32),
            pltpu.SemaphoreType.DMA,
        ],
    )


@functools.lru_cache(maxsize=None)
def _build(n):
    nc, ns = _sc_info()
    nw = nc * ns
    pw = n // nw
    nchunk = pw // _C
    nf = _N_LEVELS * _F
    mesh = plsc.VectorSubcoreMesh(
        core_axis_name="c", subcore_axis_name="s", num_cores=nc, num_subcores=ns
    )

    def body(xh, tabh, gridsh, outh, xb, wb, idxb0, idxb1, rowsb0, rowsb1,
             outb, g0, g1, g2, g3, g4, g5, sem0, sem1, osem):
        wid = lax.axis_index("c") * ns + lax.axis_index("s")
        iota = lax.iota(jnp.int32, 16)
        lane_nf = iota * nf
        lane3 = iota * 3
        idxbs = (idxb0, idxb1)
        rowsbs = (rowsb0, rowsb1)
        sems = (sem0, sem1)
        grids = (g0, g1, g2, g3, g4, g5)

        def cell(xv, gs):
            q = (xv + 1.0) / gs
            bi = q.astype(jnp.int32)
            bf = bi.astype(jnp.float32)
            minv = bf * gs - 1.0
            maxv = minv + gs
            w = (xv - minv) / (maxv - minv)
            return bi, w

        def hash_words(gx, gy, gz, ioff):
            h = gx ^ (gy * jnp.int32(_P1)) ^ (gz * jnp.int32(_P2))
            hm = h & jnp.int32(_MASK)
            t = hm & jnp.int32(127)
            return ((hm - t) << 1) + t + jnp.int32(ioff)

        def build_grids():
            for lvl in range(_N_GRID):
                pltpu.sync_copy(
                    gridsh.at[pl.ds(_GOFF[lvl], _GPAD[lvl])], grids[lvl]
                )

        def idx_pass(lvl, par):
            gs = _GS[lvl]
            ioff = lvl * _T * _F

            def g_body(g, _):
                o = g * 16
                cs = []
                for d in range(3):
                    xv = plsc.load_gather(xb, [lane3 + (o * 3 + d)])
                    bi, w = cell(xv, gs)
                    wb[par, d, pl.ds(o, 16)] = w
                    cs.append(bi)
                c0, c1, c2 = cs
                c1p = c1 * jnp.int32(_P1)
                c2p = c2 * jnp.int32(_P2)
                combos = ((c0, c0 + jnp.int32(1)),
                          (c1p, c1p + jnp.int32(_P1)),
                          (c2p, c2p + jnp.int32(_P2)))
                for j in range(8):
                    h = (combos[0][(j >> 2) & 1] ^ combos[1][(j >> 1) & 1]
                         ^ combos[2][j & 1])
                    hm = h & jnp.int32(_MASK)
                    t = hm & jnp.int32(127)
                    p0 = ((hm - t) << 1) + t + jnp.int32(ioff)
                    idxbs[par][pl.ds(j * _C + o, 16)] = p0
                    idxbs[par][pl.ds(_W + j * _C + o, 16)] = p0 + 128
                return 0

            lax.fori_loop(0, _C // 16, g_body, 0)

        def start_gather(par):
            return pltpu.async_copy(
                tabh.at[idxbs[par]], rowsbs[par], sems[par]
            )

        def trilerp_store(v, w0, w1, w2, o, lvl):
            obase = ((o >> 7) << 10) + (o & 127)
            for f in range(2):
                m = v[f]
                m = [m[2 * a] + w2 * (m[2 * a + 1] - m[2 * a])
                     for a in range(4)]
                m = [m[2 * a] + w1 * (m[2 * a + 1] - m[2 * a])
                     for a in range(2)]
                r = m[0] + w0 * (m[1] - m[0])
                ff = 2 * lvl + f
                outb[pl.ds(obase + ((ff >> 3) * _C * 8 + (ff & 7) * 128),
                           16)] = r

        def interp_pass(lvl, par):
            def g_body(g, _):
                o = g * 16
                w0 = wb[par, 0, pl.ds(o, 16)]
                w1 = wb[par, 1, pl.ds(o, 16)]
                w2 = wb[par, 2, pl.ds(o, 16)]
                v = [[rowsbs[par][pl.ds(f * _W + j * _C + o, 16)]
                      for j in range(8)] for f in range(2)]
                trilerp_store(v, w0, w1, w2, o, lvl)
                return 0

            lax.fori_loop(0, _C // 16, g_body, 0)

        def grid_pass(lvl):
            bmin, side, nvp = _GRIDS[lvl]
            s2 = side * side
            gs = _GS[lvl]
            gref = grids[lvl]
            lvbase = bmin * (s2 + side + 1)

            def g_body(g, _):
                o = g * 16
                bis, ws = [], []
                for d in range(3):
                    xv = plsc.load_gather(xb, [lane3 + (o * 3 + d)])
                    bi, w = cell(xv, gs)
                    bis.append(bi)
                    ws.append(w)
                lv = (bis[0] * jnp.int32(s2) + bis[1] * jnp.int32(side)
                      + bis[2] - jnp.int32(lvbase))
                v = [[None] * 8, [None] * 8]
                for j in range(8):
                    cj = (((j >> 2) & 1) * s2 + ((j >> 1) & 1) * side
                          + (j & 1))
                    v[0][j] = plsc.load_gather(gref, [lv + jnp.int32(cj)])
                    v[1][j] = plsc.load_gather(
                        gref, [lv + jnp.int32(cj + nvp)]
                    )
                trilerp_store(v, ws[0], ws[1], ws[2], o, lvl)
                return 0

            lax.fori_loop(0, _C // 16, g_body, 0)

        build_grids()

        def chunk_body(c, _):
            base = wid * pw + c * _C
            pltpu.sync_copy(xh.at[pl.ds(base * 3, _C * 3)], xb)
            idx_pass(_N_GRID, 0)
            dmas = [start_gather(0), None]
            idx_pass(_N_GRID + 1, 1)
            dmas[1] = start_gather(1)

            @pl.when(c > 0)
            def _():
                for ft in range(4):
                    pltpu.make_async_copy(
                        outb.at[pl.ds(ft * _C * 8, _C * 8)],
                        outh.at[pl.ds(ft * n * 8, _C * 8)], osem
                    ).wait()

            for lvl in range(_N_GRID):
                grid_pass(lvl)
            for lvl in range(_N_GRID, _N_LEVELS):
                par = (lvl - _N_GRID) & 1
                dmas[par].wait()
                interp_pass(lvl, par)
                if lvl + 2 < _N_LEVELS:
                    idx_pass(lvl + 2, par)
                    dmas[par] = start_gather(par)
            for ft in range(4):
                pltpu.async_copy(
                    outb.at[pl.ds(ft * _C * 8, _C * 8)],
                    outh.at[pl.ds(ft * n * 8 + base * 8, _C * 8)], osem
                )
            return 0

        lax.fori_loop(0, nchunk, chunk_body, 0)
        for ft in range(4):
            pltpu.make_async_copy(
                outb.at[pl.ds(ft * _C * 8, _C * 8)],
                outh.at[pl.ds(ft * n * 8, _C * 8)], osem
            ).wait()

    grid_scratch = [pltpu.VMEM((gp,), jnp.float32) for gp in _GPAD]
    return pl.kernel(
        body,
        out_type=jax.ShapeDtypeStruct((n * nf,), jnp.float32),
        mesh=mesh,
        compiler_params=pltpu.CompilerParams(needs_layout_passes=False),
        scratch_types=[
            pltpu.VMEM((3 * _C,), jnp.float32),
            pltpu.VMEM((2, 3, _C), jnp.float32),
            pltpu.VMEM((2 * _W,), jnp.int32),
            pltpu.VMEM((2 * _W,), jnp.int32),
            pltpu.VMEM((2 * _W,), jnp.float32),
            pltpu.VMEM((2 * _W,), jnp.float32),
            pltpu.VMEM((_C * nf,), jnp.float32),
            *grid_scratch,
            pltpu.SemaphoreType.DMA,
            pltpu.SemaphoreType.DMA,
            pltpu.SemaphoreType.DMA,
        ],
    )


def kernel(x, tables):
    n = x.shape[0]
    xf = x.reshape(n * 3)
    tab = (
        tables.reshape(_N_LEVELS, _T // 128, 128, _F)
        .transpose(0, 1, 3, 2)
        .reshape(_N_LEVELS * _T * _F)
    )
    grids = _build_grid_kernel()(tab)
    out = _build(n)(xf, tab, grids)
    return (
        out.reshape(4, n // 128, 8, 128)
        .transpose(1, 3, 0, 2)
        .reshape(n, _N_LEVELS * _F)
    )
